# baseline (speedup 1.0000x reference)
import jax
import jax.numpy as jnp
from jax.experimental import pallas as pl
from jax.experimental.pallas import tpu as pltpu

N1, N2 = 25, 10
DIN = 128
H0, H1 = 256, 128
P = 1 + N1 + N1 * N2
BB = 64


def _act(x):
    return jnp.where(x >= 0, x, 0.01 * x)


def _dot(a, b):
    return jnp.dot(a, b, preferred_element_type=jnp.float32)


def _tower(f_ref, w1_ref, b1_ref, w2_ref, b2_ref):
    def lane(k):
        return f_ref[:, k * DIN:(k + 1) * DIN]

    h0 = lane(0)
    h1_chunks = [lane(1 + n1) for n1 in range(N1)]
    acc0 = h1_chunks[0]
    for n1 in range(1, N1):
        acc0 = acc0 + h1_chunks[n1]
    neigh0 = acc0 * (1.0 / N1)
    ns_chunks = []
    for n1 in range(N1):
        base = 1 + N1 + n1 * N2
        s = lane(base)
        for n2 in range(1, N2):
            s = s + lane(base + n2)
        ns_chunks.append(s * (1.0 / N2))
    x1 = jnp.concatenate(h1_chunks, axis=0)
    ns = jnp.concatenate(ns_chunks, axis=0)
    w1 = w1_ref[...]
    w1a, w1b = w1[:DIN], w1[DIN:]
    b1 = b1_ref[...]
    h1n = _act(_dot(x1, w1a) + _dot(ns, w1b) + b1)
    accn = h1n[0:BB]
    for n1 in range(1, N1):
        accn = accn + h1n[n1 * BB:(n1 + 1) * BB]
    neigh = accn * (1.0 / N1)
    h0n = _act(_dot(h0, w1a) + _dot(neigh0, w1b) + b1)
    w2 = w2_ref[...]
    w2a, w2b = w2[:H0], w2[H0:]
    h0f = _act(_dot(h0n, w2a) + _dot(neigh, w2b) + b2_ref[...])
    return _act(h0f)


def _fused_kernel(uf_ref, if_ref, w1u_ref, b1u_ref, w2u_ref, b2u_ref,
                  w1i_ref, b1i_ref, w2i_ref, b2i_ref, wl_ref, bl_ref,
                  out_ref):
    uh = _tower(uf_ref, w1u_ref, b1u_ref, w2u_ref, b2u_ref)
    ih = _tower(if_ref, w1i_ref, b1i_ref, w2i_ref, b2i_ref)
    pred = _dot(uh * ih, wl_ref[...]) + bl_ref[...]
    out_ref[...] = jax.nn.sigmoid(pred)


def kernel(sampling_user_feat, sampling_item_feat, W1_u, b1_u, W2_u, b2_u,
           W1_i, b1_i, W2_i, b2_i, W_lin, b_lin):
    b = sampling_user_feat.shape[0]
    uf = sampling_user_feat.reshape(b, P * DIN)
    itf = sampling_item_feat.reshape(b, P * DIN)
    grid = (b // BB,)
    feat_spec = pl.BlockSpec((BB, P * DIN), lambda i: (i, 0))
    w1_spec = pl.BlockSpec((2 * DIN, H0), lambda i: (0, 0))
    b1_spec = pl.BlockSpec((1, H0), lambda i: (0, 0))
    w2_spec = pl.BlockSpec((2 * H0, H1), lambda i: (0, 0))
    b2_spec = pl.BlockSpec((1, H1), lambda i: (0, 0))
    wl_spec = pl.BlockSpec((H1, 2), lambda i: (0, 0))
    bl_spec = pl.BlockSpec((1, 2), lambda i: (0, 0))
    out = pl.pallas_call(
        _fused_kernel,
        grid=grid,
        in_specs=[feat_spec, feat_spec,
                  w1_spec, b1_spec, w2_spec, b2_spec,
                  w1_spec, b1_spec, w2_spec, b2_spec,
                  wl_spec, bl_spec],
        out_specs=pl.BlockSpec((BB, 2), lambda i: (i, 0)),
        out_shape=jax.ShapeDtypeStruct((b, 2), jnp.float32),
        compiler_params=pltpu.CompilerParams(
            dimension_semantics=("parallel",)),
    )(uf, itf,
      W1_u, b1_u.reshape(1, H0), W2_u, b2_u.reshape(1, H1),
      W1_i, b1_i.reshape(1, H0), W2_i, b2_i.reshape(1, H1),
      W_lin, b_lin.reshape(1, 2))
    return out


# speedup vs baseline: 1.3643x; 1.3643x over previous
import jax
import jax.numpy as jnp
from jax.experimental import pallas as pl
from jax.experimental.pallas import tpu as pltpu

N1, N2 = 25, 10
DIN = 128
H0, H1 = 256, 128
P = 1 + N1 + N1 * N2
BB = 64


def _act(x):
    return jnp.where(x >= 0, x, 0.01 * x)


def _dot(a, b):
    return jnp.dot(a, b, preferred_element_type=jnp.float32)


def _tower(f_ref, w1_ref, b1_ref, w2_ref, b2_ref):
    def lane(k):
        return f_ref[:, k, :]

    h0 = lane(0)
    h1_chunks = [lane(1 + n1) for n1 in range(N1)]
    acc0 = h1_chunks[0]
    for n1 in range(1, N1):
        acc0 = acc0 + h1_chunks[n1]
    neigh0 = acc0 * (1.0 / N1)
    ns_chunks = []
    for n1 in range(N1):
        base = 1 + N1 + n1 * N2
        s = lane(base)
        for n2 in range(1, N2):
            s = s + lane(base + n2)
        ns_chunks.append(s * (1.0 / N2))
    x1 = jnp.concatenate(h1_chunks, axis=0)
    ns = jnp.concatenate(ns_chunks, axis=0)
    w1 = w1_ref[...]
    w1a, w1b = w1[:DIN], w1[DIN:]
    b1 = b1_ref[...]
    h1n = _act(_dot(x1, w1a) + _dot(ns, w1b) + b1)
    accn = h1n[0:BB]
    for n1 in range(1, N1):
        accn = accn + h1n[n1 * BB:(n1 + 1) * BB]
    neigh = accn * (1.0 / N1)
    h0n = _act(_dot(h0, w1a) + _dot(neigh0, w1b) + b1)
    w2 = w2_ref[...]
    w2a, w2b = w2[:H0], w2[H0:]
    h0f = _act(_dot(h0n, w2a) + _dot(neigh, w2b) + b2_ref[...])
    return _act(h0f)


def _fused_kernel(uf_ref, if_ref, w1u_ref, b1u_ref, w2u_ref, b2u_ref,
                  w1i_ref, b1i_ref, w2i_ref, b2i_ref, wl_ref, bl_ref,
                  out_ref):
    uh = _tower(uf_ref, w1u_ref, b1u_ref, w2u_ref, b2u_ref)
    ih = _tower(if_ref, w1i_ref, b1i_ref, w2i_ref, b2i_ref)
    pred = _dot(uh * ih, wl_ref[...]) + bl_ref[...]
    out_ref[...] = jax.nn.sigmoid(pred)


def kernel(sampling_user_feat, sampling_item_feat, W1_u, b1_u, W2_u, b2_u,
           W1_i, b1_i, W2_i, b2_i, W_lin, b_lin):
    b = sampling_user_feat.shape[0]
    grid = (b // BB,)
    feat_spec = pl.BlockSpec((BB, P, DIN), lambda i: (i, 0, 0))
    w1_spec = pl.BlockSpec((2 * DIN, H0), lambda i: (0, 0))
    b1_spec = pl.BlockSpec((1, H0), lambda i: (0, 0))
    w2_spec = pl.BlockSpec((2 * H0, H1), lambda i: (0, 0))
    b2_spec = pl.BlockSpec((1, H1), lambda i: (0, 0))
    wl_spec = pl.BlockSpec((H1, 2), lambda i: (0, 0))
    bl_spec = pl.BlockSpec((1, 2), lambda i: (0, 0))
    out = pl.pallas_call(
        _fused_kernel,
        grid=grid,
        in_specs=[feat_spec, feat_spec,
                  w1_spec, b1_spec, w2_spec, b2_spec,
                  w1_spec, b1_spec, w2_spec, b2_spec,
                  wl_spec, bl_spec],
        out_specs=pl.BlockSpec((BB, 2), lambda i: (i, 0)),
        out_shape=jax.ShapeDtypeStruct((b, 2), jnp.float32),
        compiler_params=pltpu.CompilerParams(
            dimension_semantics=("parallel",)),
    )(sampling_user_feat, sampling_item_feat,
      W1_u, b1_u.reshape(1, H0), W2_u, b2_u.reshape(1, H1),
      W1_i, b1_i.reshape(1, H0), W2_i, b2_i.reshape(1, H1),
      W_lin, b_lin.reshape(1, 2))
    return out


# speedup vs baseline: 1.3668x; 1.0018x over previous
import jax
import jax.numpy as jnp
from jax.experimental import pallas as pl
from jax.experimental.pallas import tpu as pltpu

N1, N2 = 25, 10
DIN = 128
H0, H1 = 256, 128
P = 1 + N1 + N1 * N2
BB = 32


def _act(x):
    return jnp.where(x >= 0, x, 0.01 * x)


def _dot(a, b):
    return jnp.dot(a, b, preferred_element_type=jnp.float32)


def _tower(f_ref, w1_ref, b1_ref, w2_ref, b2_ref):
    def lane(k):
        return f_ref[:, k, :]

    h0 = lane(0)
    h1_chunks = [lane(1 + n1) for n1 in range(N1)]
    acc0 = h1_chunks[0]
    for n1 in range(1, N1):
        acc0 = acc0 + h1_chunks[n1]
    neigh0 = acc0 * (1.0 / N1)
    ns_chunks = []
    for n1 in range(N1):
        base = 1 + N1 + n1 * N2
        s = lane(base)
        for n2 in range(1, N2):
            s = s + lane(base + n2)
        ns_chunks.append(s * (1.0 / N2))
    x1 = jnp.concatenate(h1_chunks, axis=0)
    ns = jnp.concatenate(ns_chunks, axis=0)
    w1 = w1_ref[...]
    w1a, w1b = w1[:DIN], w1[DIN:]
    b1 = b1_ref[...]
    h1n = _act(_dot(x1, w1a) + _dot(ns, w1b) + b1)
    accn = h1n[0:BB]
    for n1 in range(1, N1):
        accn = accn + h1n[n1 * BB:(n1 + 1) * BB]
    neigh = accn * (1.0 / N1)
    h0n = _act(_dot(h0, w1a) + _dot(neigh0, w1b) + b1)
    w2 = w2_ref[...]
    w2a, w2b = w2[:H0], w2[H0:]
    h0f = _act(_dot(h0n, w2a) + _dot(neigh, w2b) + b2_ref[...])
    return _act(h0f)


def _fused_kernel(uf_ref, if_ref, w1u_ref, b1u_ref, w2u_ref, b2u_ref,
                  w1i_ref, b1i_ref, w2i_ref, b2i_ref, wl_ref, bl_ref,
                  out_ref):
    uh = _tower(uf_ref, w1u_ref, b1u_ref, w2u_ref, b2u_ref)
    ih = _tower(if_ref, w1i_ref, b1i_ref, w2i_ref, b2i_ref)
    pred = _dot(uh * ih, wl_ref[...]) + bl_ref[...]
    out_ref[...] = jax.nn.sigmoid(pred)


def kernel(sampling_user_feat, sampling_item_feat, W1_u, b1_u, W2_u, b2_u,
           W1_i, b1_i, W2_i, b2_i, W_lin, b_lin):
    b = sampling_user_feat.shape[0]
    grid = (b // BB,)
    feat_spec = pl.BlockSpec((BB, P, DIN), lambda i: (i, 0, 0))
    w1_spec = pl.BlockSpec((2 * DIN, H0), lambda i: (0, 0))
    b1_spec = pl.BlockSpec((1, H0), lambda i: (0, 0))
    w2_spec = pl.BlockSpec((2 * H0, H1), lambda i: (0, 0))
    b2_spec = pl.BlockSpec((1, H1), lambda i: (0, 0))
    wl_spec = pl.BlockSpec((H1, 2), lambda i: (0, 0))
    bl_spec = pl.BlockSpec((1, 2), lambda i: (0, 0))
    out = pl.pallas_call(
        _fused_kernel,
        grid=grid,
        in_specs=[feat_spec, feat_spec,
                  w1_spec, b1_spec, w2_spec, b2_spec,
                  w1_spec, b1_spec, w2_spec, b2_spec,
                  wl_spec, bl_spec],
        out_specs=pl.BlockSpec((BB, 2), lambda i: (i, 0)),
        out_shape=jax.ShapeDtypeStruct((b, 2), jnp.float32),
        compiler_params=pltpu.CompilerParams(
            dimension_semantics=("parallel",)),
    )(sampling_user_feat, sampling_item_feat,
      W1_u, b1_u.reshape(1, H0), W2_u, b2_u.reshape(1, H1),
      W1_i, b1_i.reshape(1, H0), W2_i, b2_i.reshape(1, H1),
      W_lin, b_lin.reshape(1, 2))
    return out
